# initial kernel scaffold (unmeasured)
import jax
import jax.numpy as jnp
from jax import lax
from jax.experimental import pallas as pl
from jax.experimental.pallas import tpu as pltpu


def kernel(
    x,
):
    def body(*refs):
        pass

    out_shape = jax.ShapeDtypeStruct(..., jnp.float32)
    return pl.pallas_call(body, out_shape=out_shape)(...)


# baseline (device time: 15643 ns/iter reference)
import jax
import jax.numpy as jnp
from jax import lax
from jax.experimental import pallas as pl
from jax.experimental.pallas import tpu as pltpu

Z = 4


def kernel(x):
    m, n = x.shape

    def body(x_ref, out_ref, recv_buf, send_sems, recv_sems):
        my_x = lax.axis_index("x")
        my_y = lax.axis_index("y")
        my_z = lax.axis_index("z")

        barrier_sem = pltpu.get_barrier_semaphore()
        for k in range(1, Z):
            pl.semaphore_signal(
                barrier_sem,
                inc=1,
                device_id=(my_x, my_y, (my_z + k) % Z),
                device_id_type=pl.DeviceIdType.MESH,
            )
        pl.semaphore_wait(barrier_sem, Z - 1)

        rdmas = []
        for k in range(1, Z):
            rdma = pltpu.make_async_remote_copy(
                src_ref=x_ref,
                dst_ref=recv_buf.at[k - 1],
                send_sem=send_sems.at[k - 1],
                recv_sem=recv_sems.at[k - 1],
                device_id=(my_x, my_y, (my_z + k) % Z),
                device_id_type=pl.DeviceIdType.MESH,
            )
            rdma.start()
            rdmas.append(rdma)
        for rdma in rdmas:
            rdma.wait()

        out_ref[...] = (
            x_ref[...]
            + recv_buf[0, :, :]
            + recv_buf[1, :, :]
            + recv_buf[2, :, :]
        )

    return pl.pallas_call(
        body,
        out_shape=jax.ShapeDtypeStruct((m, n), x.dtype),
        in_specs=[pl.BlockSpec(memory_space=pltpu.VMEM)],
        out_specs=pl.BlockSpec(memory_space=pltpu.VMEM),
        scratch_shapes=[
            pltpu.VMEM((Z - 1, m, n), x.dtype),
            pltpu.SemaphoreType.DMA((Z - 1,)),
            pltpu.SemaphoreType.DMA((Z - 1,)),
        ],
        compiler_params=pltpu.CompilerParams(collective_id=0),
    )(x)


# device time: 14070 ns/iter; 1.1118x vs baseline; 1.1118x over previous
import jax
import jax.numpy as jnp
from jax import lax
from jax.experimental import pallas as pl
from jax.experimental.pallas import tpu as pltpu

Z = 4
QROWS = 64


def kernel(x):
    m, n = x.shape

    def body(x_ref, out_ref, zbuf, z_send, z_recv, xy_send, xy_recv):
        my_x = lax.axis_index("x")
        my_y = lax.axis_index("y")
        my_z = lax.axis_index("z")
        q_off = (2 * my_x + my_y) * QROWS

        barrier_sem = pltpu.get_barrier_semaphore()
        for k in range(1, Z):
            pl.semaphore_signal(
                barrier_sem, inc=1,
                device_id=(my_x, my_y, (my_z + k) % Z),
                device_id_type=pl.DeviceIdType.MESH,
            )
        for dst in ((1 - my_x, my_y), (my_x, 1 - my_y), (1 - my_x, 1 - my_y)):
            pl.semaphore_signal(
                barrier_sem, inc=1,
                device_id=(dst[0], dst[1], my_z),
                device_id_type=pl.DeviceIdType.MESH,
            )
        pl.semaphore_wait(barrier_sem, 6)

        z_rdmas = []
        for k in range(1, Z):
            rdma = pltpu.make_async_remote_copy(
                src_ref=x_ref.at[pl.ds(q_off, QROWS), :],
                dst_ref=zbuf.at[k - 1],
                send_sem=z_send.at[k - 1],
                recv_sem=z_recv.at[k - 1],
                device_id=(my_x, my_y, (my_z + k) % Z),
                device_id_type=pl.DeviceIdType.MESH,
            )
            rdma.start()
            z_rdmas.append(rdma)
        for rdma in z_rdmas:
            rdma.wait()

        out_ref[pl.ds(q_off, QROWS), :] = (
            x_ref[pl.ds(q_off, QROWS), :]
            + zbuf[0, :, :]
            + zbuf[1, :, :]
            + zbuf[2, :, :]
        )

        xy_rdmas = []
        for i, dst in enumerate(
            ((1 - my_x, my_y), (my_x, 1 - my_y), (1 - my_x, 1 - my_y))
        ):
            rdma = pltpu.make_async_remote_copy(
                src_ref=out_ref.at[pl.ds(q_off, QROWS), :],
                dst_ref=out_ref.at[pl.ds(q_off, QROWS), :],
                send_sem=xy_send.at[i],
                recv_sem=xy_recv.at[i],
                device_id=(dst[0], dst[1], my_z),
                device_id_type=pl.DeviceIdType.MESH,
            )
            rdma.start()
            xy_rdmas.append(rdma)
        for rdma in xy_rdmas:
            rdma.wait()

    return pl.pallas_call(
        body,
        out_shape=jax.ShapeDtypeStruct((m, n), x.dtype),
        in_specs=[pl.BlockSpec(memory_space=pltpu.VMEM)],
        out_specs=pl.BlockSpec(memory_space=pltpu.VMEM),
        scratch_shapes=[
            pltpu.VMEM((Z - 1, QROWS, n), x.dtype),
            pltpu.SemaphoreType.DMA((Z - 1,)),
            pltpu.SemaphoreType.DMA((Z - 1,)),
            pltpu.SemaphoreType.DMA((3,)),
            pltpu.SemaphoreType.DMA((3,)),
        ],
        compiler_params=pltpu.CompilerParams(collective_id=0),
    )(x)


# device time: 13858 ns/iter; 1.1288x vs baseline; 1.0153x over previous
import jax
import jax.numpy as jnp
from jax import lax
from jax.experimental import pallas as pl
from jax.experimental.pallas import tpu as pltpu

Z = 4
QROWS = 64
HROWS = 32


def kernel(x):
    m, n = x.shape

    def body(x_ref, out_ref, zbuf, z_send, z_recv, xy_send, xy_recv):
        my_x = lax.axis_index("x")
        my_y = lax.axis_index("y")
        my_z = lax.axis_index("z")
        q_off = (2 * my_x + my_y) * QROWS
        xy_peers = ((1 - my_x, 1 - my_y), (1 - my_x, my_y), (my_x, 1 - my_y))

        barrier_sem = pltpu.get_barrier_semaphore()
        for k in range(1, Z):
            pl.semaphore_signal(
                barrier_sem, inc=1,
                device_id=(my_x, my_y, (my_z + k) % Z),
                device_id_type=pl.DeviceIdType.MESH,
            )
        for dst in xy_peers:
            pl.semaphore_signal(
                barrier_sem, inc=1,
                device_id=(dst[0], dst[1], my_z),
                device_id_type=pl.DeviceIdType.MESH,
            )
        pl.semaphore_wait(barrier_sem, 6)

        z_rdmas = [[None] * (Z - 1) for _ in range(2)]
        for h in range(2):
            for k in range(Z - 1, 0, -1):
                rdma = pltpu.make_async_remote_copy(
                    src_ref=x_ref.at[pl.ds(q_off + h * HROWS, HROWS), :],
                    dst_ref=zbuf.at[h, k - 1],
                    send_sem=z_send.at[h, k - 1],
                    recv_sem=z_recv.at[h, k - 1],
                    device_id=(my_x, my_y, (my_z + k) % Z),
                    device_id_type=pl.DeviceIdType.MESH,
                )
                rdma.start()
                z_rdmas[h][k - 1] = rdma
        xy_rdmas = []
        for h in range(2):
            for rdma in z_rdmas[h]:
                rdma.wait_recv()
            h_off = q_off + h * HROWS
            out_ref[pl.ds(h_off, HROWS), :] = (
                x_ref[pl.ds(h_off, HROWS), :]
                + zbuf[h, 0, :, :]
                + zbuf[h, 1, :, :]
                + zbuf[h, 2, :, :]
            )
            for i, dst in enumerate(xy_peers):
                rdma = pltpu.make_async_remote_copy(
                    src_ref=out_ref.at[pl.ds(h_off, HROWS), :],
                    dst_ref=out_ref.at[pl.ds(h_off, HROWS), :],
                    send_sem=xy_send.at[h, i],
                    recv_sem=xy_recv.at[h, i],
                    device_id=(dst[0], dst[1], my_z),
                    device_id_type=pl.DeviceIdType.MESH,
                )
                rdma.start()
                xy_rdmas.append(rdma)

        for rdma in xy_rdmas:
            rdma.wait()
        for half in z_rdmas:
            for rdma in half:
                rdma.wait_send()

    return pl.pallas_call(
        body,
        out_shape=jax.ShapeDtypeStruct((m, n), x.dtype),
        in_specs=[pl.BlockSpec(memory_space=pltpu.VMEM)],
        out_specs=pl.BlockSpec(memory_space=pltpu.VMEM),
        scratch_shapes=[
            pltpu.VMEM((2, Z - 1, HROWS, n), x.dtype),
            pltpu.SemaphoreType.DMA((2, Z - 1)),
            pltpu.SemaphoreType.DMA((2, Z - 1)),
            pltpu.SemaphoreType.DMA((2, 3)),
            pltpu.SemaphoreType.DMA((2, 3)),
        ],
        compiler_params=pltpu.CompilerParams(collective_id=0),
    )(x)
